# Initial kernel scaffold; baseline (speedup 1.0000x reference)
import math

import jax
import jax.numpy as jnp
from jax.experimental import pallas as pl
from jax.experimental.pallas import tpu as pltpu

_CLASSES = 32000
_SMOOTHING = 0.2
_CONF = 1.0 - _SMOOTHING
_EPS = _SMOOTHING / (_CLASSES - 2)
_C0 = _CONF * math.log(_CONF) + _SMOOTHING * math.log(_EPS)

_BR = 512
_BC = 6400


def _body(pred_ref, targ_ref, out_ref):
    j = pl.program_id(1)
    block = pred_ref[...]
    targ = targ_ref[...]
    w = (targ != 0).astype(jnp.float32)
    cols = jax.lax.broadcasted_iota(jnp.int32, (_BR, _BC), 1) + j * _BC
    is_t = (cols == targ).astype(jnp.float32)
    coeff = (_EPS - _CONF) * is_t - _EPS
    rowsum = jnp.sum(block * coeff, axis=1, keepdims=True)
    partial = jnp.sum(rowsum * w)

    @pl.when(j == 0)
    def _():
        p0 = block[:, 0:1]
        out_ref[0, 0] = partial + jnp.sum((_C0 + _EPS * p0) * w)

    @pl.when(j != 0)
    def _():
        out_ref[0, 0] += partial


def kernel(pred, target):
    n, c = pred.shape
    targ2d = target.reshape(n, 1).astype(jnp.int32)
    grid = (n // _BR, c // _BC)
    partials = pl.pallas_call(
        _body,
        grid=grid,
        in_specs=[
            pl.BlockSpec((_BR, _BC), lambda i, j: (i, j)),
            pl.BlockSpec((_BR, 1), lambda i, j: (i, 0)),
        ],
        out_specs=pl.BlockSpec((1, 1), lambda i, j: (i, 0),
                               memory_space=pltpu.SMEM),
        out_shape=jax.ShapeDtypeStruct((grid[0], 1), jnp.float32),
        compiler_params=pltpu.CompilerParams(
            dimension_semantics=("parallel", "arbitrary")),
    )(pred, targ2d)
    return jnp.sum(partials)


# speedup vs baseline: 8.0620x; 8.0620x over previous
import math

import jax
import jax.numpy as jnp
from jax.experimental import pallas as pl
from jax.experimental.pallas import tpu as pltpu

_CLASSES = 32000
_SMOOTHING = 0.2
_CONF = 1.0 - _SMOOTHING
_EPS = _SMOOTHING / (_CLASSES - 2)
_C0 = _CONF * math.log(_CONF) + _SMOOTHING * math.log(_EPS)

_BR = 512
_BC = 6400


def _body(pred_ref, targ_ref, out_ref):
    j = pl.program_id(1)
    block = pred_ref[...]
    targ = targ_ref[...]
    w = (targ != 0).astype(jnp.float32)
    cols = jax.lax.broadcasted_iota(jnp.int32, (_BR, _BC), 1) + j * _BC
    is_t = (cols == targ).astype(jnp.float32)
    coeff = (_EPS - _CONF) * is_t - _EPS
    rowsum = jnp.sum(block * coeff, axis=1, keepdims=True)
    partial = jnp.sum(rowsum * w)

    @pl.when(j == 0)
    def _():
        p0 = block[:, 0:1]
        out_ref[0, 0, 0] = partial + jnp.sum((_C0 + _EPS * p0) * w)

    @pl.when(j != 0)
    def _():
        out_ref[0, 0, 0] += partial


def kernel(pred, target):
    n, c = pred.shape
    targ2d = target.reshape(n, 1).astype(jnp.int32)
    grid = (n // _BR, c // _BC)
    partials = pl.pallas_call(
        _body,
        grid=grid,
        in_specs=[
            pl.BlockSpec((_BR, _BC), lambda i, j: (i, j)),
            pl.BlockSpec((_BR, 1), lambda i, j: (i, 0)),
        ],
        out_specs=pl.BlockSpec((1, 1, 1), lambda i, j: (i, 0, 0),
                               memory_space=pltpu.SMEM),
        out_shape=jax.ShapeDtypeStruct((grid[0], 1, 1), jnp.float32),
        compiler_params=pltpu.CompilerParams(
            dimension_semantics=("parallel", "arbitrary")),
    )(pred, targ2d)
    return jnp.sum(partials)


# speedup vs baseline: 8.2543x; 1.0238x over previous
import math

import jax
import jax.numpy as jnp
from jax.experimental import pallas as pl
from jax.experimental.pallas import tpu as pltpu

_CLASSES = 32000
_SMOOTHING = 0.2
_CONF = 1.0 - _SMOOTHING
_EPS = _SMOOTHING / (_CLASSES - 2)
_C0 = _CONF * math.log(_CONF) + _SMOOTHING * math.log(_EPS)

_BR = 1024
_BC = 6400


def _body(pred_ref, targ_ref, out_ref):
    j = pl.program_id(1)
    block = pred_ref[...]
    targ = targ_ref[...]
    w = (targ != 0).astype(jnp.float32)
    cols = jax.lax.broadcasted_iota(jnp.int32, (_BR, _BC), 1) + j * _BC
    is_t = (cols == targ).astype(jnp.float32)
    coeff = (_EPS - _CONF) * is_t - _EPS
    rowsum = jnp.sum(block * coeff, axis=1, keepdims=True)
    partial = jnp.sum(rowsum * w)

    @pl.when(j == 0)
    def _():
        p0 = block[:, 0:1]
        out_ref[0, 0, 0] = partial + jnp.sum((_C0 + _EPS * p0) * w)

    @pl.when(j != 0)
    def _():
        out_ref[0, 0, 0] += partial


def kernel(pred, target):
    n, c = pred.shape
    targ2d = target.reshape(n, 1).astype(jnp.int32)
    grid = (n // _BR, c // _BC)
    partials = pl.pallas_call(
        _body,
        grid=grid,
        in_specs=[
            pl.BlockSpec((_BR, _BC), lambda i, j: (i, j)),
            pl.BlockSpec((_BR, 1), lambda i, j: (i, 0)),
        ],
        out_specs=pl.BlockSpec((1, 1, 1), lambda i, j: (i, 0, 0),
                               memory_space=pltpu.SMEM),
        out_shape=jax.ShapeDtypeStruct((grid[0], 1, 1), jnp.float32),
        compiler_params=pltpu.CompilerParams(
            dimension_semantics=("parallel", "arbitrary")),
    )(pred, targ2d)
    return jnp.sum(partials)
